# baseline (speedup 1.0000x reference)
import jax
import jax.numpy as jnp
from jax.experimental import pallas as pl
from jax.experimental.pallas import tpu as pltpu

_K = 1024
_W = 64
_TB = 512


def _vq_block(x_ref, k_ref, xl_ref, xd_ref, stats_ref):
    xb = x_ref[0]
    k = k_ref[...]

    kxm2 = jax.lax.dot_general(
        (k * -2.0).astype(jnp.bfloat16), xb.astype(jnp.bfloat16),
        (((1,), (0,)), ((), ())),
        preferred_element_type=jnp.float32)
    x2 = jnp.sum(xb * xb, axis=0, keepdims=True)
    kk2 = jnp.sum(k * k, axis=1, keepdims=True)
    d = (x2 + kxm2) + kk2

    mind = jnp.min(d, axis=0, keepdims=True)
    iota = jax.lax.broadcasted_iota(jnp.int32, (_K, _TB), 0)
    cand = jnp.where(d == mind, iota, _K)
    midx = jnp.min(cand, axis=0)

    onehot = (iota == midx[None, :]).astype(jnp.bfloat16)
    xd = jax.lax.dot_general(
        k.astype(jnp.bfloat16), onehot, (((0,), (0,)), ((), ())),
        preferred_element_type=jnp.float32)

    xl_ref[0] = midx.reshape(1, _TB)
    xd_ref[0] = xd
    stats_ref[0, 0, 0] = jnp.sum(mind)
    stats_ref[0, 0, 1] = jnp.sum(xb)
    stats_ref[0, 0, 2] = jnp.sum(x2)


def kernel(x, k):
    N, W, T = x.shape
    gt = T // _TB
    grid = (N, gt)
    xl3, xd, stats = pl.pallas_call(
        _vq_block,
        grid=grid,
        in_specs=[
            pl.BlockSpec((1, W, _TB), lambda n, t: (n, 0, t)),
            pl.BlockSpec((_K, W), lambda n, t: (0, 0)),
        ],
        out_specs=[
            pl.BlockSpec((1, 1, _TB), lambda n, t: (n, 0, t)),
            pl.BlockSpec((1, W, _TB), lambda n, t: (n, 0, t)),
            pl.BlockSpec((1, 1, 3), lambda n, t: (n * gt + t, 0, 0),
                         memory_space=pltpu.SMEM),
        ],
        out_shape=[
            jax.ShapeDtypeStruct((N, 1, T), jnp.int32),
            jax.ShapeDtypeStruct((N, W, T), jnp.float32),
            jax.ShapeDtypeStruct((N * gt, 1, 3), jnp.float32),
        ],
        compiler_params=pltpu.CompilerParams(
            dimension_semantics=("parallel", "parallel")),
    )(x, k)

    numel = N * W * T
    ntok = N * T
    x_l = xl3.reshape(N, T)
    s = jnp.sum(stats.reshape(-1, 3), axis=0)
    fit = s[0] / ntok
    commit_loss = s[0] / numel
    mean = s[1] / numel
    prenorm = jnp.sqrt(jnp.maximum(s[2] / numel - mean * mean, 0.0))
    return (x_l, xd, commit_loss, fit, prenorm)


# speedup vs baseline: 1.1544x; 1.1544x over previous
import jax
import jax.numpy as jnp
from jax.experimental import pallas as pl
from jax.experimental.pallas import tpu as pltpu

_K = 1024
_W = 64
_TB = 512


def _vq_block(x_ref, k_ref, xl_ref, xd_ref, stats_ref):
    xb = x_ref[0]
    k = k_ref[...]

    kxm2 = jax.lax.dot_general(
        (k * -2.0).astype(jnp.bfloat16), xb.astype(jnp.bfloat16),
        (((1,), (0,)), ((), ())),
        preferred_element_type=jnp.float32)
    x2 = jnp.sum(xb * xb, axis=0, keepdims=True)
    kk2 = jnp.sum(k * k, axis=1, keepdims=True)
    d = (x2 + kxm2) + kk2

    midx = jnp.argmin(d, axis=0)
    mind = jnp.min(d, axis=0)

    onehot = (jax.lax.broadcasted_iota(jnp.int32, (_K, _TB), 0)
              == midx[None, :]).astype(jnp.bfloat16)
    xd = jax.lax.dot_general(
        k.astype(jnp.bfloat16), onehot, (((0,), (0,)), ((), ())),
        preferred_element_type=jnp.float32)

    xl_ref[0] = midx.reshape(1, _TB)
    xd_ref[0] = xd
    stats_ref[0, 0, 0] = jnp.sum(mind)
    stats_ref[0, 0, 1] = jnp.sum(xb)
    stats_ref[0, 0, 2] = jnp.sum(x2)


def kernel(x, k):
    N, W, T = x.shape
    gt = T // _TB
    grid = (N, gt)
    xl3, xd, stats = pl.pallas_call(
        _vq_block,
        grid=grid,
        in_specs=[
            pl.BlockSpec((1, W, _TB), lambda n, t: (n, 0, t)),
            pl.BlockSpec((_K, W), lambda n, t: (0, 0)),
        ],
        out_specs=[
            pl.BlockSpec((1, 1, _TB), lambda n, t: (n, 0, t)),
            pl.BlockSpec((1, W, _TB), lambda n, t: (n, 0, t)),
            pl.BlockSpec((1, 1, 3), lambda n, t: (n * gt + t, 0, 0),
                         memory_space=pltpu.SMEM),
        ],
        out_shape=[
            jax.ShapeDtypeStruct((N, 1, T), jnp.int32),
            jax.ShapeDtypeStruct((N, W, T), jnp.float32),
            jax.ShapeDtypeStruct((N * gt, 1, 3), jnp.float32),
        ],
        compiler_params=pltpu.CompilerParams(
            dimension_semantics=("parallel", "parallel")),
    )(x, k)

    numel = N * W * T
    ntok = N * T
    x_l = xl3.reshape(N, T)
    s = jnp.sum(stats.reshape(-1, 3), axis=0)
    fit = s[0] / ntok
    commit_loss = s[0] / numel
    mean = s[1] / numel
    prenorm = jnp.sqrt(jnp.maximum(s[2] / numel - mean * mean, 0.0))
    return (x_l, xd, commit_loss, fit, prenorm)
